# baseline (speedup 1.0000x reference)
import functools

import jax
import jax.numpy as jnp
from jax import lax
from jax.experimental import pallas as pl
from jax.experimental.pallas import tpu as pltpu
from jax.experimental.pallas import tpu_sc as plsc

N = 10000
E = 320000
D = 128
EPS = 1e-6

NC = 2
NS = 16
NW = NC * NS
EPT = E // NW
CHUNK = 128
NCHUNK = 80
EPT_P = NCHUNK * CHUNK
ACC_ROWS = 10240
RPT = ACC_ROWS // NS

BLK = 400


def _layer_norm_block(t, g, b):
    mu = jnp.mean(t, axis=-1, keepdims=True)
    var = jnp.mean((t - mu) ** 2, axis=-1, keepdims=True)
    return g * (t - mu) * lax.rsqrt(var + EPS) + b


def _ffn(x, w1, b1, w2, b2):
    h = jnp.maximum(jnp.dot(x, w1, preferred_element_type=jnp.float32) + b1, 0.0)
    return jnp.dot(h, w2, preferred_element_type=jnp.float32) + b2


def _ffn_body(x_ref, w1_ref, b1_ref, w2_ref, b2_ref, g_ref, b_ref, o_ref):
    x = x_ref[...]
    f = _ffn(x, w1_ref[...], b1_ref[...], w2_ref[...], b2_ref[...])
    o_ref[...] = _layer_norm_block(f + x, g_ref[...], b_ref[...])


def _ffn_ln(x, w1, b1, w2, b2, g, b):
    row = pl.BlockSpec((BLK, D), lambda i: (i, 0))
    full = pl.BlockSpec((D, D), lambda i: (0, 0))
    vec = pl.BlockSpec((1, D), lambda i: (0, 0))
    return pl.pallas_call(
        _ffn_body,
        grid=(N // BLK,),
        in_specs=[row, full, vec, full, vec, vec, vec],
        out_specs=row,
        out_shape=jax.ShapeDtypeStruct((N, D), jnp.float32),
    )(x, w1, b1, w2, b2, g, b)


def _gcn(p0, p1, ff, w, b, g, bb):
    agg = p0 + p1
    attn = jnp.maximum(jnp.dot(agg, w, preferred_element_type=jnp.float32) + b, 0.0)
    return _layer_norm_block(attn + ff, g, bb)


def _gcn_body(p0_ref, p1_ref, ff_ref, w_ref, b_ref, g_ref, bb_ref, o_ref):
    o_ref[...] = _gcn(p0_ref[0], p1_ref[0], ff_ref[...], w_ref[...], b_ref[...],
                      g_ref[...], bb_ref[...])


def _gcn_ffn_body(p0_ref, p1_ref, ff_ref, w_ref, b_ref, g_ref, bb_ref,
                  w1_ref, b1_ref, w2_ref, b2_ref, fg_ref, fb_ref, o_ref):
    out = _gcn(p0_ref[0], p1_ref[0], ff_ref[...], w_ref[...], b_ref[...],
               g_ref[...], bb_ref[...])
    f = _ffn(out, w1_ref[...], b1_ref[...], w2_ref[...], b2_ref[...])
    o_ref[...] = _layer_norm_block(f + out, fg_ref[...], fb_ref[...])


_P0 = pl.BlockSpec((1, BLK, D), lambda i: (0, i, 0))
_P1 = pl.BlockSpec((1, BLK, D), lambda i: (1, i, 0))
_ROW = pl.BlockSpec((BLK, D), lambda i: (i, 0))
_FULL = pl.BlockSpec((D, D), lambda i: (0, 0))
_VEC = pl.BlockSpec((1, D), lambda i: (0, 0))


def _gcn_ln(partials, ff_out, w, b, g, bb):
    return pl.pallas_call(
        _gcn_body,
        grid=(N // BLK,),
        in_specs=[_P0, _P1, _ROW, _FULL, _VEC, _VEC, _VEC],
        out_specs=_ROW,
        out_shape=jax.ShapeDtypeStruct((N, D), jnp.float32),
    )(partials, partials, ff_out, w, b, g, bb)


def _gcn_ln_ffn_ln(partials, ff_out, w, b, g, bb, w1, b1, w2, b2, fg, fb):
    return pl.pallas_call(
        _gcn_ffn_body,
        grid=(N // BLK,),
        in_specs=[_P0, _P1, _ROW, _FULL, _VEC, _VEC, _VEC,
                  _FULL, _VEC, _FULL, _VEC, _VEC, _VEC],
        out_specs=_ROW,
        out_shape=jax.ShapeDtypeStruct((N, D), jnp.float32),
    )(partials, partials, ff_out, w, b, g, bb, w1, b1, w2, b2, fg, fb)


def _sc_body(ff_hbm, src_hbm, dst_hbm, out_hbm,
             src_v, dbuf, rows0, rows1, zbuf, acc,
             sem0, sem1, semd0, semd1):
    cid = lax.axis_index("c")
    sid = lax.axis_index("s")
    wid = sid * NC + cid

    zero = jnp.zeros((16,), jnp.float32)
    for r in range(16):
        for c in range(D // 16):
            zbuf[r, pl.ds(c * 16, 16)] = zero
    row0 = sid * RPT

    def zero_rows(i, carry):
        pltpu.sync_copy(zbuf, acc.at[pl.ds(row0 + i * 16, 16)])
        return carry

    lax.fori_loop(0, RPT // 16, zero_rows, 0)

    pltpu.sync_copy(src_hbm.at[wid], src_v)

    def pre(j, rbuf, db, gsem, dsem):
        pltpu.async_copy(dst_hbm.at[wid, j], db, dsem)
        pltpu.async_copy(ff_hbm.at[src_v.at[j]], rbuf, gsem)

    def fin(j, rbuf, db, gsem, dsem):
        pltpu.make_async_copy(ff_hbm.at[src_v.at[j]], rbuf, gsem).wait()
        pltpu.make_async_copy(dst_hbm.at[wid, j], db, dsem).wait()
        pltpu.sync_copy(rbuf, acc.at[db], add=True)

    pre(0, rows0, dbuf.at[0], sem0, semd0)
    pre(1, rows1, dbuf.at[1], sem1, semd1)
    plsc.subcore_barrier()

    def body(g_, carry):
        j0 = 2 * g_
        fin(j0, rows0, dbuf.at[0], sem0, semd0)
        pre(j0 + 2, rows0, dbuf.at[0], sem0, semd0)
        fin(j0 + 1, rows1, dbuf.at[1], sem1, semd1)
        pre(j0 + 3, rows1, dbuf.at[1], sem1, semd1)
        return carry

    lax.fori_loop(0, NCHUNK // 2 - 1, body, 0)
    fin(NCHUNK - 2, rows0, dbuf.at[0], sem0, semd0)
    fin(NCHUNK - 1, rows1, dbuf.at[1], sem1, semd1)
    plsc.subcore_barrier()

    out_base = cid * ACC_ROWS + row0
    pltpu.sync_copy(acc.at[pl.ds(row0, RPT)], out_hbm.at[pl.ds(out_base, RPT)])


def _sc_segment_sum(ff_out, src_r, dst_r):
    mesh = plsc.VectorSubcoreMesh(core_axis_name="c", subcore_axis_name="s")
    k = functools.partial(
        pl.kernel,
        out_type=jax.ShapeDtypeStruct((NC * ACC_ROWS, D), jnp.float32),
        mesh=mesh,
        scratch_types=[
            pltpu.VMEM((NCHUNK, CHUNK), jnp.int32),
            pltpu.VMEM((2, CHUNK), jnp.int32),
            pltpu.VMEM((CHUNK, D), jnp.float32),
            pltpu.VMEM((CHUNK, D), jnp.float32),
            pltpu.VMEM((16, D), jnp.float32),
            pltpu.VMEM_SHARED((ACC_ROWS, D), jnp.float32),
            pltpu.SemaphoreType.DMA,
            pltpu.SemaphoreType.DMA,
            pltpu.SemaphoreType.DMA,
            pltpu.SemaphoreType.DMA,
        ],
    )(_sc_body)
    return k(ff_out, src_r, dst_r).reshape(NC, ACC_ROWS, D)


def kernel(features, edge_index, ff_W1, ff_b1, ff_W2, ff_b2, ffln_g, ffln_b,
           gcn_W, gcn_b, ln_g, ln_b):
    pad = EPT_P - EPT
    src_r = jnp.pad(edge_index[0].reshape(NW, EPT), ((0, 0), (0, pad)),
                    constant_values=0).reshape(NW, NCHUNK, CHUNK)
    dst_r = jnp.pad(edge_index[1].reshape(NW, EPT), ((0, 0), (0, pad)),
                    constant_values=N).reshape(NW, NCHUNK, CHUNK)
    L = ff_W1.shape[0]
    v = lambda a: a.reshape(1, D)

    ff_out = _ffn_ln(features, ff_W1[0], v(ff_b1[0]), ff_W2[0], v(ff_b2[0]),
                     v(ffln_g[0]), v(ffln_b[0]))
    for i in range(L):
        partials = _sc_segment_sum(ff_out, src_r, dst_r)
        if i + 1 < L:
            ff_out = _gcn_ln_ffn_ln(
                partials, ff_out, gcn_W[i], v(gcn_b[i]), v(ln_g[i]), v(ln_b[i]),
                ff_W1[i + 1], v(ff_b1[i + 1]), ff_W2[i + 1], v(ff_b2[i + 1]),
                v(ffln_g[i + 1]), v(ffln_b[i + 1]))
        else:
            output = _gcn_ln(partials, ff_out, gcn_W[i], v(gcn_b[i]),
                             v(ln_g[i]), v(ln_b[i]))
    return output


# speedup vs baseline: 1.6873x; 1.6873x over previous
import functools

import jax
import jax.numpy as jnp
from jax import lax
from jax.experimental import pallas as pl
from jax.experimental.pallas import tpu as pltpu
from jax.experimental.pallas import tpu_sc as plsc

N = 10000
E = 320000
D = 128
EPS = 1e-6

NC = 2
NS = 16
NW = NC * NS
EPT = E // NW
CHUNK = 80
NCHUNK = 126
EPT_P = NCHUNK * CHUNK
ACC_ROWS = 10240
RPT = ACC_ROWS // NS

BLK = 400


def _layer_norm_block(t, g, b):
    mu = jnp.mean(t, axis=-1, keepdims=True)
    var = jnp.mean((t - mu) ** 2, axis=-1, keepdims=True)
    return g * (t - mu) * lax.rsqrt(var + EPS) + b


def _ffn(x, w1, b1, w2, b2):
    h = jnp.maximum(jnp.dot(x, w1, preferred_element_type=jnp.float32) + b1, 0.0)
    return jnp.dot(h, w2, preferred_element_type=jnp.float32) + b2


def _ffn_body(x_ref, w1_ref, b1_ref, w2_ref, b2_ref, g_ref, b_ref, o_ref):
    x = x_ref[...]
    f = _ffn(x, w1_ref[...], b1_ref[...], w2_ref[...], b2_ref[...])
    o_ref[...] = _layer_norm_block(f + x, g_ref[...], b_ref[...])


def _ffn_ln(x, w1, b1, w2, b2, g, b):
    row = pl.BlockSpec((BLK, D), lambda i: (i, 0))
    full = pl.BlockSpec((D, D), lambda i: (0, 0))
    vec = pl.BlockSpec((1, D), lambda i: (0, 0))
    return pl.pallas_call(
        _ffn_body,
        grid=(N // BLK,),
        in_specs=[row, full, vec, full, vec, vec, vec],
        out_specs=row,
        out_shape=jax.ShapeDtypeStruct((N, D), jnp.float32),
    )(x, w1, b1, w2, b2, g, b)


def _gcn(p0, p1, ff, w, b, g, bb):
    agg = p0 + p1
    attn = jnp.maximum(jnp.dot(agg, w, preferred_element_type=jnp.float32) + b, 0.0)
    return _layer_norm_block(attn + ff, g, bb)


def _gcn_body(p0_ref, p1_ref, ff_ref, w_ref, b_ref, g_ref, bb_ref, o_ref):
    o_ref[...] = _gcn(p0_ref[0], p1_ref[0], ff_ref[...], w_ref[...], b_ref[...],
                      g_ref[...], bb_ref[...])


def _gcn_ffn_body(p0_ref, p1_ref, ff_ref, w_ref, b_ref, g_ref, bb_ref,
                  w1_ref, b1_ref, w2_ref, b2_ref, fg_ref, fb_ref, o_ref):
    out = _gcn(p0_ref[0], p1_ref[0], ff_ref[...], w_ref[...], b_ref[...],
               g_ref[...], bb_ref[...])
    f = _ffn(out, w1_ref[...], b1_ref[...], w2_ref[...], b2_ref[...])
    o_ref[...] = _layer_norm_block(f + out, fg_ref[...], fb_ref[...])


_P0 = pl.BlockSpec((1, BLK, D), lambda i: (0, i, 0))
_P1 = pl.BlockSpec((1, BLK, D), lambda i: (1, i, 0))
_ROW = pl.BlockSpec((BLK, D), lambda i: (i, 0))
_FULL = pl.BlockSpec((D, D), lambda i: (0, 0))
_VEC = pl.BlockSpec((1, D), lambda i: (0, 0))


def _gcn_ln(partials, ff_out, w, b, g, bb):
    return pl.pallas_call(
        _gcn_body,
        grid=(N // BLK,),
        in_specs=[_P0, _P1, _ROW, _FULL, _VEC, _VEC, _VEC],
        out_specs=_ROW,
        out_shape=jax.ShapeDtypeStruct((N, D), jnp.float32),
    )(partials, partials, ff_out, w, b, g, bb)


def _gcn_ln_ffn_ln(partials, ff_out, w, b, g, bb, w1, b1, w2, b2, fg, fb):
    return pl.pallas_call(
        _gcn_ffn_body,
        grid=(N // BLK,),
        in_specs=[_P0, _P1, _ROW, _FULL, _VEC, _VEC, _VEC,
                  _FULL, _VEC, _FULL, _VEC, _VEC, _VEC],
        out_specs=_ROW,
        out_shape=jax.ShapeDtypeStruct((N, D), jnp.float32),
    )(partials, partials, ff_out, w, b, g, bb, w1, b1, w2, b2, fg, fb)


def _sc_body(ff_hbm, src_hbm, dst_hbm, out_hbm,
             src_v, dbuf, rows0, rows1, zbuf, acc,
             sem0, sem1, semd0, semd1):
    cid = lax.axis_index("c")
    sid = lax.axis_index("s")
    wid = sid * NC + cid

    zero = jnp.zeros((16,), jnp.float32)
    for r in range(16):
        for c in range(D // 16):
            zbuf[r, pl.ds(c * 16, 16)] = zero
    row0 = sid * RPT

    def zero_rows(i, carry):
        pltpu.sync_copy(zbuf, acc.at[pl.ds(row0 + i * 16, 16)])
        return carry

    lax.fori_loop(0, RPT // 16, zero_rows, 0)

    pltpu.sync_copy(src_hbm.at[wid], src_v)

    def pre(j, rbuf, db, gsem, dsem):
        pltpu.async_copy(dst_hbm.at[wid, j], db, dsem)
        pltpu.async_copy(ff_hbm.at[src_v.at[j]], rbuf, gsem)

    def fin(j, rbuf, db, gsem, dsem):
        pltpu.make_async_copy(ff_hbm.at[src_v.at[j]], rbuf, gsem).wait()
        pltpu.make_async_copy(dst_hbm.at[wid, j], db, dsem).wait()
        pltpu.sync_copy(rbuf, acc.at[db], add=True)

    pre(0, rows0, dbuf.at[0], sem0, semd0)
    pre(1, rows1, dbuf.at[1], sem1, semd1)
    plsc.subcore_barrier()

    def body(g_, carry):
        j0 = 2 * g_
        fin(j0, rows0, dbuf.at[0], sem0, semd0)
        pre(j0 + 2, rows0, dbuf.at[0], sem0, semd0)
        fin(j0 + 1, rows1, dbuf.at[1], sem1, semd1)
        pre(j0 + 3, rows1, dbuf.at[1], sem1, semd1)
        return carry

    lax.fori_loop(0, NCHUNK // 2 - 1, body, 0)
    fin(NCHUNK - 2, rows0, dbuf.at[0], sem0, semd0)
    fin(NCHUNK - 1, rows1, dbuf.at[1], sem1, semd1)
    plsc.subcore_barrier()

    out_base = cid * ACC_ROWS + row0
    pltpu.sync_copy(acc.at[pl.ds(row0, RPT)], out_hbm.at[pl.ds(out_base, RPT)])


def _sc_segment_sum(ff_out, src_r, dst_r):
    mesh = plsc.VectorSubcoreMesh(core_axis_name="c", subcore_axis_name="s")
    k = functools.partial(
        pl.kernel,
        out_type=jax.ShapeDtypeStruct((NC * ACC_ROWS, D), jnp.float32),
        mesh=mesh,
        scratch_types=[
            pltpu.VMEM((NCHUNK, CHUNK), jnp.int32),
            pltpu.VMEM((2, CHUNK), jnp.int32),
            pltpu.VMEM((CHUNK, D), jnp.float32),
            pltpu.VMEM((CHUNK, D), jnp.float32),
            pltpu.VMEM((16, D), jnp.float32),
            pltpu.VMEM_SHARED((ACC_ROWS, D), jnp.float32),
            pltpu.SemaphoreType.DMA,
            pltpu.SemaphoreType.DMA,
            pltpu.SemaphoreType.DMA,
            pltpu.SemaphoreType.DMA,
        ],
    )(_sc_body)
    return k(ff_out, src_r, dst_r).reshape(NC, ACC_ROWS, D)


def kernel(features, edge_index, ff_W1, ff_b1, ff_W2, ff_b2, ffln_g, ffln_b,
           gcn_W, gcn_b, ln_g, ln_b):
    pad = EPT_P - EPT
    src_r = jnp.pad(edge_index[0].reshape(NW, EPT), ((0, 0), (0, pad)),
                    constant_values=0).reshape(NW, NCHUNK, CHUNK)
    dst_r = jnp.pad(edge_index[1].reshape(NW, EPT), ((0, 0), (0, pad)),
                    constant_values=N).reshape(NW, NCHUNK, CHUNK)
    L = ff_W1.shape[0]
    v = lambda a: a.reshape(1, D)

    ff_out = _ffn_ln(features, ff_W1[0], v(ff_b1[0]), ff_W2[0], v(ff_b2[0]),
                     v(ffln_g[0]), v(ffln_b[0]))
    for i in range(L):
        partials = _sc_segment_sum(ff_out, src_r, dst_r)
        if i + 1 < L:
            ff_out = _gcn_ln_ffn_ln(
                partials, ff_out, gcn_W[i], v(gcn_b[i]), v(ln_g[i]), v(ln_b[i]),
                ff_W1[i + 1], v(ff_b1[i + 1]), ff_W2[i + 1], v(ff_b2[i + 1]),
                v(ffln_g[i + 1]), v(ffln_b[i + 1]))
        else:
            output = _gcn_ln(partials, ff_out, gcn_W[i], v(gcn_b[i]),
                             v(ln_g[i]), v(ln_b[i]))
    return output


# speedup vs baseline: 2.0913x; 1.2394x over previous
import functools

import jax
import jax.numpy as jnp
from jax import lax
from jax.experimental import pallas as pl
from jax.experimental.pallas import tpu as pltpu
from jax.experimental.pallas import tpu_sc as plsc

N = 10000
E = 320000
D = 128
EPS = 1e-6

NC = 2
NS = 16
NW = NC * NS
EPT = E // NW
CHUNK = 40
NCHUNK = 250
EPT_P = NCHUNK * CHUNK
ACC_ROWS = 10240
RPT = ACC_ROWS // NS

BLK = 400


def _layer_norm_block(t, g, b):
    mu = jnp.mean(t, axis=-1, keepdims=True)
    var = jnp.mean((t - mu) ** 2, axis=-1, keepdims=True)
    return g * (t - mu) * lax.rsqrt(var + EPS) + b


def _ffn(x, w1, b1, w2, b2):
    h = jnp.maximum(jnp.dot(x, w1, preferred_element_type=jnp.float32) + b1, 0.0)
    return jnp.dot(h, w2, preferred_element_type=jnp.float32) + b2


def _ffn_body(x_ref, w1_ref, b1_ref, w2_ref, b2_ref, g_ref, b_ref, o_ref):
    x = x_ref[...]
    f = _ffn(x, w1_ref[...], b1_ref[...], w2_ref[...], b2_ref[...])
    o_ref[...] = _layer_norm_block(f + x, g_ref[...], b_ref[...])


def _ffn_ln(x, w1, b1, w2, b2, g, b):
    row = pl.BlockSpec((BLK, D), lambda i: (i, 0))
    full = pl.BlockSpec((D, D), lambda i: (0, 0))
    vec = pl.BlockSpec((1, D), lambda i: (0, 0))
    return pl.pallas_call(
        _ffn_body,
        grid=(N // BLK,),
        in_specs=[row, full, vec, full, vec, vec, vec],
        out_specs=row,
        out_shape=jax.ShapeDtypeStruct((N, D), jnp.float32),
    )(x, w1, b1, w2, b2, g, b)


def _gcn(p0, p1, ff, w, b, g, bb):
    agg = p0 + p1
    attn = jnp.maximum(jnp.dot(agg, w, preferred_element_type=jnp.float32) + b, 0.0)
    return _layer_norm_block(attn + ff, g, bb)


def _gcn_body(p0_ref, p1_ref, ff_ref, w_ref, b_ref, g_ref, bb_ref, o_ref):
    o_ref[...] = _gcn(p0_ref[0], p1_ref[0], ff_ref[...], w_ref[...], b_ref[...],
                      g_ref[...], bb_ref[...])


def _gcn_ffn_body(p0_ref, p1_ref, ff_ref, w_ref, b_ref, g_ref, bb_ref,
                  w1_ref, b1_ref, w2_ref, b2_ref, fg_ref, fb_ref, o_ref):
    out = _gcn(p0_ref[0], p1_ref[0], ff_ref[...], w_ref[...], b_ref[...],
               g_ref[...], bb_ref[...])
    f = _ffn(out, w1_ref[...], b1_ref[...], w2_ref[...], b2_ref[...])
    o_ref[...] = _layer_norm_block(f + out, fg_ref[...], fb_ref[...])


_P0 = pl.BlockSpec((1, BLK, D), lambda i: (0, i, 0))
_P1 = pl.BlockSpec((1, BLK, D), lambda i: (1, i, 0))
_ROW = pl.BlockSpec((BLK, D), lambda i: (i, 0))
_FULL = pl.BlockSpec((D, D), lambda i: (0, 0))
_VEC = pl.BlockSpec((1, D), lambda i: (0, 0))


def _gcn_ln(partials, ff_out, w, b, g, bb):
    return pl.pallas_call(
        _gcn_body,
        grid=(N // BLK,),
        in_specs=[_P0, _P1, _ROW, _FULL, _VEC, _VEC, _VEC],
        out_specs=_ROW,
        out_shape=jax.ShapeDtypeStruct((N, D), jnp.float32),
    )(partials, partials, ff_out, w, b, g, bb)


def _gcn_ln_ffn_ln(partials, ff_out, w, b, g, bb, w1, b1, w2, b2, fg, fb):
    return pl.pallas_call(
        _gcn_ffn_body,
        grid=(N // BLK,),
        in_specs=[_P0, _P1, _ROW, _FULL, _VEC, _VEC, _VEC,
                  _FULL, _VEC, _FULL, _VEC, _VEC, _VEC],
        out_specs=_ROW,
        out_shape=jax.ShapeDtypeStruct((N, D), jnp.float32),
    )(partials, partials, ff_out, w, b, g, bb, w1, b1, w2, b2, fg, fb)


def _sc_body(ff_hbm, src_hbm, dst_hbm, out_hbm,
             src_v, dbuf, rows0, rows1, zbuf, acc,
             sem0, sem1, semd0, semd1):
    cid = lax.axis_index("c")
    sid = lax.axis_index("s")
    wid = sid * NC + cid

    zero = jnp.zeros((16,), jnp.float32)
    for r in range(16):
        for c in range(D // 16):
            zbuf[r, pl.ds(c * 16, 16)] = zero
    row0 = sid * RPT

    def zero_rows(i, carry):
        pltpu.sync_copy(zbuf, acc.at[pl.ds(row0 + i * 16, 16)])
        return carry

    lax.fori_loop(0, RPT // 16, zero_rows, 0)

    pltpu.sync_copy(src_hbm.at[wid], src_v)

    def pre(j, rbuf, db, gsem, dsem):
        pltpu.async_copy(dst_hbm.at[wid, j], db, dsem)
        pltpu.async_copy(ff_hbm.at[src_v.at[j]], rbuf, gsem)

    def fin(j, rbuf, db, gsem, dsem):
        pltpu.make_async_copy(ff_hbm.at[src_v.at[j]], rbuf, gsem).wait()
        pltpu.make_async_copy(dst_hbm.at[wid, j], db, dsem).wait()
        pltpu.sync_copy(rbuf, acc.at[db], add=True)

    pre(0, rows0, dbuf.at[0], sem0, semd0)
    pre(1, rows1, dbuf.at[1], sem1, semd1)
    plsc.subcore_barrier()

    def body(g_, carry):
        j0 = 2 * g_
        fin(j0, rows0, dbuf.at[0], sem0, semd0)
        pre(j0 + 2, rows0, dbuf.at[0], sem0, semd0)
        fin(j0 + 1, rows1, dbuf.at[1], sem1, semd1)
        pre(j0 + 3, rows1, dbuf.at[1], sem1, semd1)
        return carry

    lax.fori_loop(0, NCHUNK // 2 - 1, body, 0)
    fin(NCHUNK - 2, rows0, dbuf.at[0], sem0, semd0)
    fin(NCHUNK - 1, rows1, dbuf.at[1], sem1, semd1)
    plsc.subcore_barrier()

    out_base = cid * ACC_ROWS + row0
    pltpu.sync_copy(acc.at[pl.ds(row0, RPT)], out_hbm.at[pl.ds(out_base, RPT)])


def _sc_segment_sum(ff_out, src_r, dst_r):
    mesh = plsc.VectorSubcoreMesh(core_axis_name="c", subcore_axis_name="s")
    k = functools.partial(
        pl.kernel,
        out_type=jax.ShapeDtypeStruct((NC * ACC_ROWS, D), jnp.float32),
        mesh=mesh,
        scratch_types=[
            pltpu.VMEM((NCHUNK, CHUNK), jnp.int32),
            pltpu.VMEM((2, CHUNK), jnp.int32),
            pltpu.VMEM((CHUNK, D), jnp.float32),
            pltpu.VMEM((CHUNK, D), jnp.float32),
            pltpu.VMEM((16, D), jnp.float32),
            pltpu.VMEM_SHARED((ACC_ROWS, D), jnp.float32),
            pltpu.SemaphoreType.DMA,
            pltpu.SemaphoreType.DMA,
            pltpu.SemaphoreType.DMA,
            pltpu.SemaphoreType.DMA,
        ],
    )(_sc_body)
    return k(ff_out, src_r, dst_r).reshape(NC, ACC_ROWS, D)


def kernel(features, edge_index, ff_W1, ff_b1, ff_W2, ff_b2, ffln_g, ffln_b,
           gcn_W, gcn_b, ln_g, ln_b):
    pad = EPT_P - EPT
    src_r = jnp.pad(edge_index[0].reshape(NW, EPT), ((0, 0), (0, pad)),
                    constant_values=0).reshape(NW, NCHUNK, CHUNK)
    dst_r = jnp.pad(edge_index[1].reshape(NW, EPT), ((0, 0), (0, pad)),
                    constant_values=N).reshape(NW, NCHUNK, CHUNK)
    L = ff_W1.shape[0]
    v = lambda a: a.reshape(1, D)

    ff_out = _ffn_ln(features, ff_W1[0], v(ff_b1[0]), ff_W2[0], v(ff_b2[0]),
                     v(ffln_g[0]), v(ffln_b[0]))
    for i in range(L):
        partials = _sc_segment_sum(ff_out, src_r, dst_r)
        if i + 1 < L:
            ff_out = _gcn_ln_ffn_ln(
                partials, ff_out, gcn_W[i], v(gcn_b[i]), v(ln_g[i]), v(ln_b[i]),
                ff_W1[i + 1], v(ff_b1[i + 1]), ff_W2[i + 1], v(ff_b2[i + 1]),
                v(ffln_g[i + 1]), v(ffln_b[i + 1]))
        else:
            output = _gcn_ln(partials, ff_out, gcn_W[i], v(gcn_b[i]),
                             v(ln_g[i]), v(ln_b[i]))
    return output
